# baseline (speedup 1.0000x reference)
import jax
import jax.numpy as jnp
from jax import lax
from jax.experimental import pallas as pl
from jax.experimental.pallas import tpu as pltpu
from jax.experimental.pallas import tpu_sc as plsc

D = 64
SCALE = 8.0
LANES = 16
NC, NS = 2, 16
NW = NC * NS
N_XROWS, SEQ = 4096, 200
B_PER_W = N_XROWS * SEQ // NW
XROWS_PER_W = N_XROWS // NW
CHUNK = 128
N_CH = B_PER_W // CHUNK
UNROLL = 50
XROWS_PER_STEP = UNROLL * CHUNK // SEQ


def _pieces(m):
    f = CHUNK * m
    xro, off = divmod(f, SEQ)
    n1 = min(SEQ - off, CHUNK)
    out = [(0, xro, off, n1)]
    if n1 < CHUNK:
        out.append((n1, xro + 1, 0, CHUNK - n1))
    return out


def _emb_body(jdx_hbm, poff_hbm, t2_hbm, out_hbm,
              jdx_v, poff_v, g0, g1, s0, s1,
              gsem0, gsem1, osem0, osem1):
    wid = lax.axis_index("s") * NC + lax.axis_index("c")
    base = wid * B_PER_W
    xrow_base = wid * XROWS_PER_W

    gbufs = (g0, g1)
    sbufs = (s0, s1)
    gsems = (gsem0, gsem1)
    osems = (osem0, osem1)

    pltpu.sync_copy(jdx_hbm.at[pl.ds(base, B_PER_W)], jdx_v)
    pltpu.sync_copy(poff_hbm.at[pl.ds(base, B_PER_W)], poff_v)

    def fire_gather(c, b):
        pltpu.async_copy(t2_hbm.at[jdx_v.at[pl.ds(c * CHUNK, CHUNK)]],
                         gbufs[b], gsems[b])

    def wait_gather(b):
        pltpu.make_async_copy(t2_hbm.at[pl.ds(0, CHUNK), :], gbufs[b],
                              gsems[b]).wait()

    def scale_buf(c, b):
        g = gbufs[b]
        s = sbufs[b]
        po = poff_v

        @plsc.parallel_loop(0, CHUNK // LANES, 1, unroll=1)
        def _(g16):
            t0 = g16 * LANES
            pv = po[pl.ds(c * CHUNK + t0, LANES)]
            for k in range(LANES):
                off = pv[k]
                for l in range(D // LANES):
                    s[0, t0 + k, pl.ds(l * LANES, LANES)] = (
                        g[t0 + k, pl.ds(off + l * LANES, LANES)] * SCALE)

    def fire_store(c, b):
        pltpu.async_copy(sbufs[b],
                         out_hbm.at[pl.ds(base // CHUNK + c, 1), :, :],
                         osems[b])

    def wait_store(b):
        pltpu.make_async_copy(sbufs[b], out_hbm.at[pl.ds(0, 1), :, :],
                              osems[b]).wait()

    fire_gather(jnp.int32(0), 0)
    fire_gather(jnp.int32(1), 1)

    def pair_body(p, carry):
        for b in (0, 1):
            c = 2 * p + b
            wait_gather(b)
            scale_buf(c, b)

            @pl.when(c >= 2)
            def _():
                wait_store(b)

            fire_store(c, b)

            @pl.when(c + 2 < N_CH)
            def _():
                fire_gather(c + 2, b)
        return carry

    lax.fori_loop(0, N_CH // 2, pair_body, 0)

    wait_store(0)
    wait_store(1)


@jax.jit
def kernel(x, table):
    xi = x.reshape(-1).astype(jnp.int32)
    jdx = lax.shift_right_logical(xi, 1)
    poff = (xi & 1) * D
    t2 = table.reshape(500000, 2 * D)
    run = pl.kernel(
        _emb_body,
        out_type=jax.ShapeDtypeStruct((N_XROWS * SEQ // CHUNK, CHUNK, D), jnp.float32),
        mesh=plsc.VectorSubcoreMesh(core_axis_name="c", subcore_axis_name="s"),
        scratch_types=[
            pltpu.VMEM((B_PER_W,), jnp.int32),
            pltpu.VMEM((B_PER_W,), jnp.int32),
            pltpu.VMEM((CHUNK, 2 * D), jnp.float32),
            pltpu.VMEM((CHUNK, 2 * D), jnp.float32),
            pltpu.VMEM((1, CHUNK, D), jnp.float32),
            pltpu.VMEM((1, CHUNK, D), jnp.float32),
            pltpu.SemaphoreType.DMA,
            pltpu.SemaphoreType.DMA,
            pltpu.SemaphoreType.DMA,
            pltpu.SemaphoreType.DMA,
        ],
        compiler_params=pltpu.CompilerParams(use_tc_tiling_on_sc=True),
    )
    out = run(jdx, poff, t2)
    return out.reshape(x.shape[0], x.shape[1], D)


# speedup vs baseline: 1.0003x; 1.0003x over previous
import jax
import jax.numpy as jnp
from jax import lax
from jax.experimental import pallas as pl
from jax.experimental.pallas import tpu as pltpu
from jax.experimental.pallas import tpu_sc as plsc

D = 64
SCALE = 8.0
LANES = 16
NC, NS = 2, 16
NW = NC * NS
N_XROWS, SEQ = 4096, 200
B_PER_W = N_XROWS * SEQ // NW
XROWS_PER_W = N_XROWS // NW
CHUNK = 128
N_CH = B_PER_W // CHUNK
UNROLL = 50
XROWS_PER_STEP = UNROLL * CHUNK // SEQ


def _pieces(m):
    f = CHUNK * m
    xro, off = divmod(f, SEQ)
    n1 = min(SEQ - off, CHUNK)
    out = [(0, xro, off, n1)]
    if n1 < CHUNK:
        out.append((n1, xro + 1, 0, CHUNK - n1))
    return out


def _emb_body(jdx_hbm, poff_hbm, t2_hbm, out_hbm,
              jdx_v, poff_v, g0, g1, s0, s1, po_sm,
              gsem0, gsem1, osem0, osem1):
    wid = lax.axis_index("s") * NC + lax.axis_index("c")
    base = wid * B_PER_W
    xrow_base = wid * XROWS_PER_W

    gbufs = (g0, g1)
    sbufs = (s0, s1)
    gsems = (gsem0, gsem1)
    osems = (osem0, osem1)

    pltpu.sync_copy(jdx_hbm.at[pl.ds(base, B_PER_W)], jdx_v)
    pltpu.sync_copy(poff_hbm.at[pl.ds(base, B_PER_W)], poff_v)

    def fire_gather(c, b):
        pltpu.async_copy(t2_hbm.at[jdx_v.at[pl.ds(c * CHUNK, CHUNK)]],
                         gbufs[b], gsems[b])

    def wait_gather(b):
        pltpu.make_async_copy(t2_hbm.at[pl.ds(0, CHUNK), :], gbufs[b],
                              gsems[b]).wait()

    def scale_buf(c, b):
        g = gbufs[b]
        s = sbufs[b]

        @plsc.parallel_loop(0, CHUNK // LANES, 1, unroll=1)
        def _(g16):
            t0 = g16 * LANES
            pv = poff_v[pl.ds(c * CHUNK + t0, LANES)]
            for k in range(LANES):
                po_sm[b, t0 + k] = pv[k]

        @plsc.parallel_loop(0, CHUNK, 1, unroll=1)
        def _(t):
            off = po_sm[b, t]
            for l in range(D // LANES):
                s[0, t, pl.ds(l * LANES, LANES)] = (
                    g[t, pl.ds(off + l * LANES, LANES)] * SCALE)

    def fire_store(xrow0, b, m):
        for (sr, xro, so, n) in _pieces(m):
            pltpu.async_copy(
                sbufs[b].at[:, pl.ds(sr, n), :],
                out_hbm.at[pl.ds(xrow0 + xro, 1), pl.ds(so, n), :],
                osems[b])

    def wait_store(b):
        pltpu.make_async_copy(sbufs[b],
                              out_hbm.at[pl.ds(0, 1), pl.ds(0, CHUNK), :],
                              osems[b]).wait()

    fire_gather(jnp.int32(0), 0)
    fire_gather(jnp.int32(1), 1)

    def step_body(ss, carry):
        xrow0 = xrow_base + ss * XROWS_PER_STEP
        for m in range(UNROLL):
            c = ss * UNROLL + m
            b = m % 2
            wait_gather(b)
            scale_buf(c, b)

            @pl.when(c >= 2)
            def _():
                wait_store(b)

            fire_store(xrow0, b, m)

            @pl.when(c + 2 < N_CH)
            def _():
                fire_gather(c + 2, b)
        return carry

    lax.fori_loop(0, N_CH // UNROLL, step_body, 0)

    wait_store(0)
    wait_store(1)


@jax.jit
def kernel(x, table):
    xi = x.reshape(-1).astype(jnp.int32)
    jdx = lax.shift_right_logical(xi, 1)
    poff = (xi & 1) * D
    t2 = table.reshape(500000, 2 * D)
    run = pl.kernel(
        _emb_body,
        out_type=jax.ShapeDtypeStruct((N_XROWS, SEQ, D), jnp.float32),
        mesh=plsc.VectorSubcoreMesh(core_axis_name="c", subcore_axis_name="s"),
        scratch_types=[
            pltpu.VMEM((B_PER_W,), jnp.int32),
            pltpu.VMEM((B_PER_W,), jnp.int32),
            pltpu.VMEM((CHUNK, 2 * D), jnp.float32),
            pltpu.VMEM((CHUNK, 2 * D), jnp.float32),
            pltpu.VMEM((1, CHUNK, D), jnp.float32),
            pltpu.VMEM((1, CHUNK, D), jnp.float32),
            pltpu.SMEM((2, CHUNK), jnp.int32),
            pltpu.SemaphoreType.DMA,
            pltpu.SemaphoreType.DMA,
            pltpu.SemaphoreType.DMA,
            pltpu.SemaphoreType.DMA,
        ],
        compiler_params=pltpu.CompilerParams(use_tc_tiling_on_sc=True),
    )
    return run(jdx, poff, t2)
